# baseline (speedup 1.0000x reference)
import functools

import jax
import jax.numpy as jnp
from jax import lax
from jax.experimental import pallas as pl
from jax.experimental.pallas import tpu as pltpu
from jax.experimental.pallas import tpu_sc as plsc

BATCH = 16384
NFEAT = 4096
NCOLS = 64

NCORES = 2
NSUB = 16
NWORKERS = NCORES * NSUB
ROWS_PER_W = BATCH // NWORKERS
LANES = 16
CVR = NCOLS // LANES

SUB = 8
LANES_TC = 128
TILE_WORDS = SUB * LANES_TC
TROW_WORDS = (NFEAT // LANES_TC) * TILE_WORDS
TROWS_PER_W = ROWS_PER_W // SUB

GROUP = 64
NGROUPS = ROWS_PER_W // GROUP
RING = 3
LAG = 1
GROUP_WORDS = GROUP * LANES_TC


@functools.partial(
    pl.kernel,
    out_type=jax.ShapeDtypeStruct((BATCH * LANES_TC,), jnp.float32),
    mesh=plsc.VectorSubcoreMesh(core_axis_name="c", subcore_axis_name="s"),
    scratch_types=[
        pltpu.VMEM((NCOLS,), jnp.int32),
        pltpu.VMEM((ROWS_PER_W, NCOLS), jnp.int32),
        pltpu.VMEM((RING * GROUP_WORDS,), jnp.float32),
        pltpu.SemaphoreType.DMA,
        pltpu.SemaphoreType.DMA,
    ],
)
def _sc_gather_cols(x_hbm, cols_hbm, out_hbm, colv, idxv, ring, gsem, osem):
    wid = lax.axis_index("s") * NCORES + lax.axis_index("c")
    row0 = wid * ROWS_PER_W
    trow0 = row0 // SUB

    pltpu.sync_copy(cols_hbm, colv)

    cts = []
    for m in range(CVR):
        c = colv[pl.ds(m * LANES, LANES)]
        cts.append(c + lax.shift_right_logical(c, 7) * (TILE_WORDS - LANES_TC))
    cts = tuple(cts)

    def build_fire_group(g, ct):
        base = (g % RING) * GROUP_WORDS
        for rl in range(GROUP):
            r = g * GROUP + rl
            t = r // SUB
            sub = rl % SUB
            stripe = (trow0 + t) * TROW_WORDS
            for m in range(CVR):
                idxv[r, pl.ds(m * LANES, LANES)] = (
                    ct[m] + (stripe + sub * LANES_TC)
                )
            pltpu.async_copy(
                x_hbm.at[idxv.at[r]],
                ring.at[pl.ds(base + rl * LANES_TC, NCOLS)],
                gsem,
            )
        return ct

    def drain_copy_group(g):
        base = (g % RING) * GROUP_WORDS
        for rl in range(GROUP):
            r = g * GROUP + rl
            pltpu.make_async_copy(
                x_hbm.at[idxv.at[r]],
                ring.at[pl.ds(base + rl * LANES_TC, NCOLS)],
                gsem,
            ).wait()
        pltpu.async_copy(
            ring.at[pl.ds(base, GROUP_WORDS)],
            out_hbm.at[pl.ds((row0 + g * GROUP) * LANES_TC, GROUP_WORDS)],
            osem,
        )

    def wait_one_out_copy():
        pltpu.make_async_copy(
            ring.at[pl.ds(0, GROUP_WORDS)],
            out_hbm.at[pl.ds(row0 * LANES_TC, GROUP_WORDS)],
            osem,
        ).wait()

    ct = cts
    for g in range(RING):
        ct = build_fire_group(g, ct)
        if g >= LAG:
            drain_copy_group(g - LAG)

    def step(g, ct):
        wait_one_out_copy()
        ct = build_fire_group(g, ct)
        drain_copy_group(g - LAG)
        return ct

    ct = lax.fori_loop(RING, NGROUPS, step, ct)

    for g in range(NGROUPS - LAG, NGROUPS):
        drain_copy_group(g)
    for _ in range(RING):
        wait_one_out_copy()


def kernel(x, column_indices):
    x_tiles = (
        x.reshape(BATCH // SUB, SUB, NFEAT // LANES_TC, LANES_TC)
        .swapaxes(1, 2)
        .reshape(-1)
    )
    out_pad = _sc_gather_cols(x_tiles, column_indices)
    return out_pad.reshape(BATCH, LANES_TC)[:, :NCOLS]


# speedup vs baseline: 1.0260x; 1.0260x over previous
import functools

import jax
import jax.numpy as jnp
from jax import lax
from jax.experimental import pallas as pl
from jax.experimental.pallas import tpu as pltpu
from jax.experimental.pallas import tpu_sc as plsc

BATCH = 16384
NFEAT = 4096
NCOLS = 64

NCORES = 2
NSUB = 16
NWORKERS = NCORES * NSUB
ROWS_PER_W = BATCH // NWORKERS
LANES = 16
CVR = NCOLS // LANES

SUB = 8
LANES_TC = 128
TILE_WORDS = SUB * LANES_TC
TROW_WORDS = (NFEAT // LANES_TC) * TILE_WORDS
TROWS_PER_W = ROWS_PER_W // SUB

GROUP = 32
NGROUPS = ROWS_PER_W // GROUP
RING = 4
LAG = 2
GROUP_WORDS = GROUP * LANES_TC


@functools.partial(
    pl.kernel,
    out_type=jax.ShapeDtypeStruct((BATCH * LANES_TC,), jnp.float32),
    mesh=plsc.VectorSubcoreMesh(core_axis_name="c", subcore_axis_name="s"),
    scratch_types=[
        pltpu.VMEM((NCOLS,), jnp.int32),
        pltpu.VMEM((ROWS_PER_W, NCOLS), jnp.int32),
        pltpu.VMEM((RING * GROUP_WORDS,), jnp.float32),
        pltpu.SemaphoreType.DMA,
        pltpu.SemaphoreType.DMA,
    ],
)
def _sc_gather_cols(x_hbm, cols_hbm, out_hbm, colv, idxv, ring, gsem, osem):
    wid = lax.axis_index("s") * NCORES + lax.axis_index("c")
    row0 = wid * ROWS_PER_W
    trow0 = row0 // SUB

    pltpu.sync_copy(cols_hbm, colv)

    cts = []
    for m in range(CVR):
        c = colv[pl.ds(m * LANES, LANES)]
        cts.append(c + lax.shift_right_logical(c, 7) * (TILE_WORDS - LANES_TC))
    cts = tuple(cts)

    def build_fire_group(g, ct):
        base = (g % RING) * GROUP_WORDS
        for rl in range(GROUP):
            r = g * GROUP + rl
            t = r // SUB
            sub = rl % SUB
            stripe = (trow0 + t) * TROW_WORDS
            for m in range(CVR):
                idxv[r, pl.ds(m * LANES, LANES)] = (
                    ct[m] + (stripe + sub * LANES_TC)
                )
            pltpu.async_copy(
                x_hbm.at[idxv.at[r]],
                ring.at[pl.ds(base + rl * LANES_TC, NCOLS)],
                gsem,
            )
        return ct

    def drain_copy_group(g):
        base = (g % RING) * GROUP_WORDS
        for rl in range(GROUP):
            r = g * GROUP + rl
            pltpu.make_async_copy(
                x_hbm.at[idxv.at[r]],
                ring.at[pl.ds(base + rl * LANES_TC, NCOLS)],
                gsem,
            ).wait()
        pltpu.async_copy(
            ring.at[pl.ds(base, GROUP_WORDS)],
            out_hbm.at[pl.ds((row0 + g * GROUP) * LANES_TC, GROUP_WORDS)],
            osem,
        )

    def wait_one_out_copy():
        pltpu.make_async_copy(
            ring.at[pl.ds(0, GROUP_WORDS)],
            out_hbm.at[pl.ds(row0 * LANES_TC, GROUP_WORDS)],
            osem,
        ).wait()

    ct = cts
    for g in range(RING):
        ct = build_fire_group(g, ct)
        if g >= LAG:
            drain_copy_group(g - LAG)

    def step(g, ct):
        wait_one_out_copy()
        ct = build_fire_group(g, ct)
        drain_copy_group(g - LAG)
        return ct

    ct = lax.fori_loop(RING, NGROUPS, step, ct)

    for g in range(NGROUPS - LAG, NGROUPS):
        drain_copy_group(g)
    for _ in range(RING):
        wait_one_out_copy()


def kernel(x, column_indices):
    x_tiles = (
        x.reshape(BATCH // SUB, SUB, NFEAT // LANES_TC, LANES_TC)
        .swapaxes(1, 2)
        .reshape(-1)
    )
    out_pad = _sc_gather_cols(x_tiles, column_indices)
    return out_pad.reshape(BATCH, LANES_TC)[:, :NCOLS]
